# Initial kernel scaffold; baseline (speedup 1.0000x reference)
import jax
import jax.numpy as jnp
from jax.experimental import pallas as pl


def kernel(x, position_embedding_table):
    raise NotImplementedError("write your pallas kernel here")


# speedup vs baseline: 3.5476x; 3.5476x over previous
import functools

import jax
import jax.numpy as jnp
from jax import lax
from jax.experimental import pallas as pl
from jax.experimental.pallas import tpu as pltpu
from jax.experimental.pallas import tpu_sc as plsc

_EMBED = 64
_NW = 32
_CHUNK = 128


@functools.cache
def _build(n_rows, x_size):
    per_w = n_rows // _NW
    nch = per_w // _CHUNK
    mesh = plsc.VectorSubcoreMesh(core_axis_name="c", subcore_axis_name="s")

    @functools.partial(
        pl.kernel,
        mesh=mesh,
        compiler_params=pltpu.CompilerParams(use_tc_tiling_on_sc=False),
        out_type=jax.ShapeDtypeStruct((n_rows, _EMBED), jnp.float32),
        scratch_types=[
            pltpu.VMEM((nch, _CHUNK), jnp.int32),
            pltpu.VMEM((_CHUNK, _EMBED), jnp.float32),
            pltpu.SemaphoreType.DMA,
        ],
    )
    def gather_kernel(idx_hbm, table_hbm, out_hbm, idx_v, buf, sem):
        wid = lax.axis_index("s") * 2 + lax.axis_index("c")
        base = wid * per_w
        pltpu.sync_copy(idx_hbm.at[wid], idx_v)

        def body(j, carry):
            pltpu.async_copy(table_hbm.at[idx_v.at[j]], buf, sem).wait()
            pltpu.sync_copy(buf, out_hbm.at[pl.ds(base + j * _CHUNK, _CHUNK)])
            return carry

        lax.fori_loop(0, nch, body, 0)

    return gather_kernel


def kernel(x, position_embedding_table):
    b, h = x.shape
    n = b * h
    idx = x.astype(jnp.int32).reshape(_NW, n // (_NW * _CHUNK), _CHUNK)
    fn = _build(n, position_embedding_table.shape[0])
    out = fn(idx, position_embedding_table)
    return out.reshape(b, h, _EMBED)


# speedup vs baseline: 4.2429x; 1.1960x over previous
import functools

import jax
import jax.numpy as jnp
from jax import lax
from jax.experimental import pallas as pl
from jax.experimental.pallas import tpu as pltpu
from jax.experimental.pallas import tpu_sc as plsc

_EMBED = 64
_NW = 32
_CHUNK = 128
_K = 4
_GROUP = _K * _CHUNK


@functools.cache
def _build(n_rows, x_size):
    per_w = n_rows // _NW
    ngroups = per_w // _GROUP
    assert ngroups % 2 == 0
    mesh = plsc.VectorSubcoreMesh(core_axis_name="c", subcore_axis_name="s")

    @functools.partial(
        pl.kernel,
        mesh=mesh,
        compiler_params=pltpu.CompilerParams(use_tc_tiling_on_sc=False),
        out_type=jax.ShapeDtypeStruct((n_rows, _EMBED), jnp.float32),
        scratch_types=[
            pltpu.VMEM((per_w // _CHUNK, _CHUNK), jnp.int32),
            pltpu.VMEM((_GROUP, _EMBED), jnp.float32),
            pltpu.VMEM((_GROUP, _EMBED), jnp.float32),
            pltpu.SemaphoreType.DMA,
            pltpu.SemaphoreType.DMA,
            pltpu.SemaphoreType.DMA,
            pltpu.SemaphoreType.DMA,
        ],
    )
    def gather_kernel(idx_hbm, table_hbm, out_hbm, idx_v, buf_a, buf_b,
                      sem_ga, sem_gb, sem_wa, sem_wb):
        wid = lax.axis_index("s") * 2 + lax.axis_index("c")
        base = wid * per_w
        pltpu.sync_copy(idx_hbm.at[wid], idx_v)

        def start_gathers(g, buf, sem):
            for b in range(_K):
                j = g * _K + b
                pltpu.async_copy(
                    table_hbm.at[idx_v.at[j]],
                    buf.at[pl.ds(b * _CHUNK, _CHUNK)], sem)

        def wait_gathers(g, buf, sem):
            for b in range(_K):
                j = g * _K + b
                pltpu.make_async_copy(
                    table_hbm.at[idx_v.at[j]],
                    buf.at[pl.ds(b * _CHUNK, _CHUNK)], sem).wait()

        def out_slice(g):
            return out_hbm.at[pl.ds(base + g * _GROUP, _GROUP)]

        start_gathers(0, buf_a, sem_ga)

        def body(i, carry):
            ga = 2 * i
            gb = 2 * i + 1

            wait_gathers(ga, buf_a, sem_ga)

            @pl.when(i > 0)
            def _():
                pltpu.make_async_copy(buf_b, out_slice(gb - 2), sem_wb).wait()

            start_gathers(gb, buf_b, sem_gb)
            pltpu.async_copy(buf_a, out_slice(ga), sem_wa)

            wait_gathers(gb, buf_b, sem_gb)
            pltpu.make_async_copy(buf_a, out_slice(ga), sem_wa).wait()

            @pl.when(ga + 2 < ngroups)
            def _():
                start_gathers(ga + 2, buf_a, sem_ga)

            pltpu.async_copy(buf_b, out_slice(gb), sem_wb)
            return carry

        lax.fori_loop(0, ngroups // 2, body, 0)
        pltpu.make_async_copy(buf_b, out_slice(ngroups - 1), sem_wb).wait()

    return gather_kernel


def kernel(x, position_embedding_table):
    b, h = x.shape
    n = b * h
    idx = x.astype(jnp.int32).reshape(_NW, n // (_NW * _CHUNK), _CHUNK)
    fn = _build(n, position_embedding_table.shape[0])
    out = fn(idx, position_embedding_table)
    return out.reshape(b, h, _EMBED)
